# baseline (speedup 1.0000x reference)
import jax
import jax.numpy as jnp
import numpy as np
from jax.experimental import pallas as pl
from jax.experimental.pallas import tpu as pltpu
from jax.experimental.shard_map import shard_map as _shard_map
from jax.sharding import PartitionSpec as P

DIM = 1024
HEADS = 8
DIM_KEY = 32
NUM_KEYS = 2048
PK = 2
K16 = 16

TILE = 256
NEG = float("-inf")


def _topk16_paired(vals, width, payload=None):
    half = width // 2
    lo, hi = vals[:, :half], vals[:, half:]
    piota = jax.lax.broadcasted_iota(jnp.int32, lo.shape, 1)
    gt = hi > lo
    vp = jnp.where(gt, hi, lo)
    vnext = jnp.where(gt, lo, hi)
    idx2 = jnp.where(gt, piota + half, piota)
    idxo = jnp.where(gt, piota, piota + half)
    if payload is not None:
        p_lo, p_hi = payload[:, :half], payload[:, half:]
        pcur = jnp.where(gt, p_hi, p_lo)
        pother = jnp.where(gt, p_lo, p_hi)
    svals, sidx, spay = [], [], []
    for t in range(K16):
        m = jnp.max(vp, axis=1, keepdims=True)
        cand = jnp.where(vp == m, idx2, width)
        j = jnp.min(cand, axis=1, keepdims=True)
        svals.append(m)
        sidx.append(j)
        pmask = cand == j
        if payload is not None:
            spay.append(jnp.sum(jnp.where(pmask, pcur, 0), axis=1, keepdims=True))
            pcur = jnp.where(pmask, pother, pcur)
        if t < K16 - 1:
            vp = jnp.where(pmask, vnext, vp)
            vnext = jnp.where(pmask, NEG, vnext)
            idx2 = jnp.where(pmask, idxo, idx2)
    out_s = jnp.concatenate(svals, axis=1)
    out_i = jnp.concatenate(sidx, axis=1)
    if payload is not None:
        return out_s, out_i, jnp.concatenate(spay, axis=1)
    return out_s, out_i


def _pk_kernel(x_ref, wt_ref, kt_ref, out_s_ref, out_i_ref):
    xt = x_ref[...]
    q = jnp.dot(xt, wt_ref[...], preferred_element_type=jnp.float32)
    for h in range(HEADS):
        ss, ii = [], []
        for p in range(PK):
            col = (p * HEADS + h) * DIM_KEY
            qp = q[:, col:col + DIM_KEY]
            sim = jnp.dot(qp, kt_ref[p, h], preferred_element_type=jnp.float32)
            s, i = _topk16_paired(sim, NUM_KEYS)
            ss.append(s)
            ii.append(i)
        i1s = ii[1] * NUM_KEYS
        blocks_s = [ss[0][:, a:a + 1] + ss[1] for a in range(K16)]
        blocks_i = [ii[0][:, a:a + 1] + i1s for a in range(K16)]
        comb_s = jnp.concatenate(blocks_s, axis=1)
        comb_i = jnp.concatenate(blocks_i, axis=1)
        fs, _, fi = _topk16_paired(comb_s, K16 * K16, payload=comb_i)
        out_s_ref[:, h * K16:(h + 1) * K16] = fs
        out_i_ref[:, h * K16:(h + 1) * K16] = fi


def _run(x2, wt, kt, interpret=False):
    n = x2.shape[0]
    return pl.pallas_call(
        _pk_kernel,
        grid=(n // TILE,),
        in_specs=[
            pl.BlockSpec((TILE, DIM), lambda i: (i, 0)),
            pl.BlockSpec((DIM, DIM_KEY * PK * HEADS), lambda i: (0, 0)),
            pl.BlockSpec((PK, HEADS, DIM_KEY, NUM_KEYS), lambda i: (0, 0, 0, 0)),
        ],
        out_specs=[
            pl.BlockSpec((TILE, HEADS * K16), lambda i: (i, 0)),
            pl.BlockSpec((TILE, HEADS * K16), lambda i: (i, 0)),
        ],
        out_shape=[
            jax.ShapeDtypeStruct((n, HEADS * K16), jnp.float32),
            jax.ShapeDtypeStruct((n, HEADS * K16), jnp.int32),
        ],
        compiler_params=pltpu.CompilerParams(
            dimension_semantics=("parallel",)),
        interpret=interpret,
    )(x2, wt, kt)


def kernel(x, W, keys):
    B, N, _ = x.shape
    x2 = x.reshape(B * N, DIM)
    wt = W.T
    kt = jnp.transpose(keys, (0, 2, 3, 1))
    mesh = jax.sharding.Mesh(np.asarray(jax.devices()[:2]), ("d",))
    sharded = _shard_map(
        lambda a, b, c: tuple(_run(a, b, c)),
        mesh=mesh,
        in_specs=(P("d", None), P(None, None), P(None, None, None, None)),
        out_specs=(P("d", None), P("d", None)),
        check_rep=False,
    )
    out_s, out_i = sharded(x2, wt, kt)
    fs = out_s.reshape(B, N, HEADS, K16)
    fi = out_i.reshape(B, N, HEADS, K16)
    return (fs, fi)


# speedup vs baseline: 1.1174x; 1.1174x over previous
import jax
import jax.numpy as jnp
import numpy as np
from jax.experimental import pallas as pl
from jax.experimental.pallas import tpu as pltpu
from jax.experimental.shard_map import shard_map as _shard_map
from jax.sharding import PartitionSpec as P

DIM = 1024
HEADS = 8
DIM_KEY = 32
NUM_KEYS = 2048
PK = 2
K16 = 16

TILE = 512
NEG = float("-inf")


def _topk16_paired(vals, width, payload=None):
    half = width // 2
    lo, hi = vals[:, :half], vals[:, half:]
    piota = jax.lax.broadcasted_iota(jnp.int32, lo.shape, 1)
    gt = hi > lo
    vp = jnp.where(gt, hi, lo)
    vnext = jnp.where(gt, lo, hi)
    idx2 = jnp.where(gt, piota + half, piota)
    idxo = jnp.where(gt, piota, piota + half)
    if payload is not None:
        p_lo, p_hi = payload[:, :half], payload[:, half:]
        pcur = jnp.where(gt, p_hi, p_lo)
        pother = jnp.where(gt, p_lo, p_hi)
    svals, sidx, spay = [], [], []
    for t in range(K16):
        m = jnp.max(vp, axis=1, keepdims=True)
        cand = jnp.where(vp == m, idx2, width)
        j = jnp.min(cand, axis=1, keepdims=True)
        svals.append(m)
        sidx.append(j)
        pmask = cand == j
        if payload is not None:
            spay.append(jnp.sum(jnp.where(pmask, pcur, 0), axis=1, keepdims=True))
            pcur = jnp.where(pmask, pother, pcur)
        if t < K16 - 1:
            vp = jnp.where(pmask, vnext, vp)
            vnext = jnp.where(pmask, NEG, vnext)
            idx2 = jnp.where(pmask, idxo, idx2)
    out_s = jnp.concatenate(svals, axis=1)
    out_i = jnp.concatenate(sidx, axis=1)
    if payload is not None:
        return out_s, out_i, jnp.concatenate(spay, axis=1)
    return out_s, out_i


def _pk_kernel(x_ref, wt_ref, kt_ref, out_s_ref, out_i_ref):
    xt = x_ref[...]
    q = jnp.dot(xt, wt_ref[...], preferred_element_type=jnp.float32)
    for h in range(HEADS):
        ss, ii = [], []
        for p in range(PK):
            col = (p * HEADS + h) * DIM_KEY
            qp = q[:, col:col + DIM_KEY]
            sim = jnp.dot(qp, kt_ref[p, h], preferred_element_type=jnp.float32)
            s, i = _topk16_paired(sim, NUM_KEYS)
            ss.append(s)
            ii.append(i)
        i1s = ii[1] * NUM_KEYS
        blocks_s = [ss[0][:, a:a + 1] + ss[1] for a in range(K16)]
        blocks_i = [ii[0][:, a:a + 1] + i1s for a in range(K16)]
        comb_s = jnp.concatenate(blocks_s, axis=1)
        comb_i = jnp.concatenate(blocks_i, axis=1)
        fs, _, fi = _topk16_paired(comb_s, K16 * K16, payload=comb_i)
        out_s_ref[:, h * K16:(h + 1) * K16] = fs
        out_i_ref[:, h * K16:(h + 1) * K16] = fi


def _run(x2, wt, kt, interpret=False):
    n = x2.shape[0]
    return pl.pallas_call(
        _pk_kernel,
        grid=(n // TILE,),
        in_specs=[
            pl.BlockSpec((TILE, DIM), lambda i: (i, 0)),
            pl.BlockSpec((DIM, DIM_KEY * PK * HEADS), lambda i: (0, 0)),
            pl.BlockSpec((PK, HEADS, DIM_KEY, NUM_KEYS), lambda i: (0, 0, 0, 0)),
        ],
        out_specs=[
            pl.BlockSpec((TILE, HEADS * K16), lambda i: (i, 0)),
            pl.BlockSpec((TILE, HEADS * K16), lambda i: (i, 0)),
        ],
        out_shape=[
            jax.ShapeDtypeStruct((n, HEADS * K16), jnp.float32),
            jax.ShapeDtypeStruct((n, HEADS * K16), jnp.int32),
        ],
        compiler_params=pltpu.CompilerParams(
            dimension_semantics=("parallel",)),
        interpret=interpret,
    )(x2, wt, kt)


def kernel(x, W, keys):
    B, N, _ = x.shape
    x2 = x.reshape(B * N, DIM)
    wt = W.T
    kt = jnp.transpose(keys, (0, 2, 3, 1))
    mesh = jax.sharding.Mesh(np.asarray(jax.devices()[:2]), ("d",))
    sharded = _shard_map(
        lambda a, b, c: tuple(_run(a, b, c)),
        mesh=mesh,
        in_specs=(P("d", None), P(None, None), P(None, None, None, None)),
        out_specs=(P("d", None), P("d", None)),
        check_rep=False,
    )
    out_s, out_i = sharded(x2, wt, kt)
    fs = out_s.reshape(B, N, HEADS, K16)
    fi = out_i.reshape(B, N, HEADS, K16)
    return (fs, fi)
